# Initial kernel scaffold; baseline (speedup 1.0000x reference)
import jax
import jax.numpy as jnp
from jax.experimental import pallas as pl


def kernel(x_capec, edge_index_parentof, edge_index_childof, edge_index_canprecede, edge_index_canfollow, edge_index_peerof, W_src, W_dst, att_src, att_dst, bias):
    raise NotImplementedError("write your pallas kernel here")


# speedup vs baseline: 47.2544x; 47.2544x over previous
import dataclasses
import functools

import jax
import jax.numpy as jnp
from jax import lax
from jax.experimental import pallas as pl
from jax.experimental.pallas import tpu as pltpu
from jax.experimental.pallas import tpu_sc as plsc

N = 10000
D = 128
H = 2
C = 128
NREL = 5
E = 320000

NPAD = 10240
NSUB = 16
EB = 128
CHUNKS_PER_SUB = 157
E_PAD = CHUNKS_PER_SUB * NSUB * EB
N_ACC = 10240
ROWS_PER_SUB = 640
LAST_ROWS = 400
BLKA = 256
BLKD = 200


def _lrelu(v):
    return jnp.where(v > 0.0, v, v * 0.2)


def _a1_body(x_ref, ws_ref, wd_ref, atts_ref, attd_ref,
             xs_out_ref, asrc_ref, adst_ref):
    x = x_ref[...]
    xs = jnp.dot(x, ws_ref[...], preferred_element_type=jnp.float32)
    xd = jnp.dot(x, wd_ref[...], preferred_element_type=jnp.float32)
    a_s, a_d = [], []
    for h in range(H):
        xs_h = xs[:, h * C:(h + 1) * C]
        xd_h = xd[:, h * C:(h + 1) * C]
        xs_out_ref[h] = xs_h
        a_s.append(jnp.sum(xs_h * atts_ref[h][None, :], axis=-1, keepdims=True))
        a_d.append(jnp.sum(xd_h * attd_ref[h][None, :], axis=-1, keepdims=True))
    asrc_ref[...] = jnp.concatenate(a_s, axis=1)
    adst_ref[...] = jnp.concatenate(a_d, axis=1)


def _a2_body(asrc_ref, adst_ref, g_ref, wself_ref):
    a_s = asrc_ref[...]
    a_d = adst_ref[...]
    g = _lrelu(jnp.max(a_s, axis=0, keepdims=True)
               + jnp.max(a_d, axis=0, keepdims=True))
    g_ref[...] = jnp.broadcast_to(g.T, (H, 16))
    wself_ref[...] = jnp.exp(_lrelu(a_s + a_d) - g)


def _d_body(num_ref, den_ref, xs_ref, wself_ref, bias_ref, out_ref):
    for h in range(H):
        xsh = xs_ref[h]
        ws = wself_ref[:, h:h + 1]
        acc = jnp.zeros((BLKD, C), jnp.float32)
        for r in range(NREL):
            numer = num_ref[h, r] + ws * xsh
            den = (jnp.sum(den_ref[h, r], axis=-1)[:, None] + ws + 1e-16)
            acc = acc + numer / den
        out_ref[:, h * C:(h + 1) * C] = acc + bias_ref[0, h * C:(h + 1) * C][None, :]


def _bcast_lane(v, j):
    idx = jnp.full((16, 1), j, jnp.int32)
    dnums = lax.GatherDimensionNumbers(
        offset_dims=(), collapsed_slice_dims=(0,), start_index_map=(0,))
    return lax.gather(v, idx, dnums, (1,),
                      mode=lax.GatherScatterMode.PROMISE_IN_BOUNDS)


def _zero_rowbuf(rowbuf):
    @pl.loop(0, EB)
    def _zz(i):
        i = i.astype(jnp.int32)
        for cc in range(C // 16):
            rowbuf.at[i, pl.ds(cc * 16, 16)][...] = jnp.zeros((16,), jnp.float32)


def _sc_body(xs_hbm, asrc_hbm, adst_hbm, g_hbm, esrc_hbm, edst_hbm,
             out_hbm, den_hbm,
             asrc_tab, adst_tab, g_tab, den_tab, srcbuf, dstbuf, rowbuf, acc, sem):
    h = lax.axis_index("c").astype(jnp.int32)
    sid = lax.axis_index("s").astype(jnp.int32)
    rows_per_sub = jnp.int32(ROWS_PER_SUB)
    chunks_per_sub = jnp.int32(CHUNKS_PER_SUB)

    pltpu.sync_copy(asrc_hbm.at[h], asrc_tab)
    pltpu.sync_copy(adst_hbm.at[h], adst_tab)
    pltpu.sync_copy(g_hbm.at[h], g_tab)

    for r in range(NREL):
        _zero_rowbuf(rowbuf)

        @pl.loop(0, N_ACC, step=16)
        def _zd(i):
            i = i.astype(jnp.int32)
            den_tab.at[pl.ds(i, 16)][...] = jnp.zeros((16,), jnp.float32)

        @pl.loop(0, ROWS_PER_SUB, step=EB)
        def _z(i):
            i = i.astype(jnp.int32)
            pltpu.sync_copy(rowbuf, acc.at[pl.ds(sid * rows_per_sub + i, EB)])

        plsc.subcore_barrier()

        @pl.loop(0, CHUNKS_PER_SUB)
        def _chunk(t):
            t = t.astype(jnp.int32)
            row = sid * chunks_per_sub + t
            pltpu.sync_copy(esrc_hbm.at[r, row], srcbuf)
            pltpu.sync_copy(edst_hbm.at[r, row], dstbuf)
            pltpu.async_copy(xs_hbm.at[h].at[srcbuf], rowbuf, sem).wait()
            gv = g_tab[pl.ds(0, 16)]

            @pl.loop(0, EB, step=16)
            def _grp(i):
                i = i.astype(jnp.int32)
                idx_s = srcbuf[pl.ds(i, 16)]
                idx_d = dstbuf[pl.ds(i, 16)]
                a_s = plsc.load_gather(asrc_tab, [idx_s])
                a_d = plsc.load_gather(adst_tab, [idx_d])
                w = jnp.exp(_lrelu(a_s + a_d) - gv)
                for j in range(16):
                    wj = _bcast_lane(w, j)
                    for cc in range(C // 16):
                        sl = pl.ds(cc * 16, 16)
                        v = rowbuf[i + j, sl]
                        rowbuf.at[i + j, sl][...] = v * wj
                    dj = _bcast_lane(idx_d, j)
                    dv = plsc.load_gather(den_tab, [dj])
                    plsc.store_scatter(den_tab, [dj], dv + wj)

            pltpu.sync_copy(rowbuf, acc.at[dstbuf], add=True)

        plsc.subcore_barrier()

        @pl.when(sid < NSUB - 1)
        def _d1():
            pltpu.sync_copy(acc.at[pl.ds(sid * rows_per_sub, ROWS_PER_SUB)],
                            out_hbm.at[h, r, pl.ds(sid * rows_per_sub, ROWS_PER_SUB)])

        @pl.when(sid == NSUB - 1)
        def _d2():
            pltpu.sync_copy(acc.at[pl.ds((NSUB - 1) * ROWS_PER_SUB, LAST_ROWS)],
                            out_hbm.at[h, r, pl.ds((NSUB - 1) * ROWS_PER_SUB, LAST_ROWS)])

        pltpu.sync_copy(den_tab, den_hbm.at[h, r, sid])

        plsc.subcore_barrier()


def _make_sc_kernel():
    mesh = plsc.VectorSubcoreMesh(core_axis_name="c", subcore_axis_name="s")
    cp = pltpu.CompilerParams()
    if "needs_layout_passes" in pltpu.CompilerParams.__dataclass_fields__:
        cp = dataclasses.replace(cp, needs_layout_passes=False)
    return functools.partial(
        pl.kernel,
        mesh=mesh,
        compiler_params=cp,
        out_type=[
            jax.ShapeDtypeStruct((H, NREL, N, C), jnp.float32),
            jax.ShapeDtypeStruct((H, NREL, NSUB, N_ACC), jnp.float32),
        ],
        scratch_types=[
            pltpu.VMEM((N + 16,), jnp.float32),
            pltpu.VMEM((N + 16,), jnp.float32),
            pltpu.VMEM((16,), jnp.float32),
            pltpu.VMEM((N_ACC,), jnp.float32),
            pltpu.VMEM((EB,), jnp.int32),
            pltpu.VMEM((EB,), jnp.int32),
            pltpu.VMEM((EB, C), jnp.float32),
            pltpu.VMEM_SHARED((N_ACC, C), jnp.float32),
            pltpu.SemaphoreType.DMA,
        ],
    )(_sc_body)


def kernel(x_capec, edge_index_parentof, edge_index_childof, edge_index_canprecede,
           edge_index_canfollow, edge_index_peerof, W_src, W_dst, att_src, att_dst, bias):
    import jax._src.config as _jcfg
    with _jcfg.enable_x64(False):
        x = x_capec.astype(jnp.float32)
        xpad = jnp.pad(x, ((0, NPAD - N), (0, 0)))

        a1 = pl.pallas_call(
            _a1_body,
            grid=(NPAD // BLKA,),
            in_specs=[
                pl.BlockSpec((BLKA, D), lambda i: (i, 0)),
                pl.BlockSpec((D, H * C), lambda i: (0, 0)),
                pl.BlockSpec((D, H * C), lambda i: (0, 0)),
                pl.BlockSpec((H, C), lambda i: (0, 0)),
                pl.BlockSpec((H, C), lambda i: (0, 0)),
            ],
            out_specs=[
                pl.BlockSpec((H, BLKA, C), lambda i: (0, i, 0)),
                pl.BlockSpec((BLKA, H), lambda i: (i, 0)),
                pl.BlockSpec((BLKA, H), lambda i: (i, 0)),
            ],
            out_shape=[
                jax.ShapeDtypeStruct((H, NPAD, C), jnp.float32),
                jax.ShapeDtypeStruct((NPAD, H), jnp.float32),
                jax.ShapeDtypeStruct((NPAD, H), jnp.float32),
            ],
        )
        xs_t, a_src, a_dst = a1(xpad, W_src.astype(jnp.float32), W_dst.astype(jnp.float32),
                                att_src.astype(jnp.float32), att_dst.astype(jnp.float32))

        a2 = pl.pallas_call(
            _a2_body,
            out_shape=[
                jax.ShapeDtypeStruct((H, 16), jnp.float32),
                jax.ShapeDtypeStruct((NPAD, H), jnp.float32),
            ],
        )
        g, wself = a2(a_src, a_dst)

        xs_n = xs_t[:, :N, :]
        pad_tab = ((0, 0), (0, 16))
        asrc_tt = jnp.pad(a_src[:N].T, pad_tab)
        adst_tt = jnp.pad(a_dst[:N].T, pad_tab)

        edges = [edge_index_parentof, edge_index_childof, edge_index_canprecede,
                 edge_index_canfollow, edge_index_peerof]
        esrc = jnp.stack([e[0] for e in edges]).astype(jnp.int32)
        edst = jnp.stack([e[1] for e in edges]).astype(jnp.int32)
        esrc = jnp.pad(esrc, ((0, 0), (0, E_PAD - E)), constant_values=0)
        edst = jnp.pad(edst, ((0, 0), (0, E_PAD - E)), constant_values=N)
        esrc = esrc.reshape(NREL, E_PAD // EB, EB)
        edst = edst.reshape(NREL, E_PAD // EB, EB)

        num, den = _make_sc_kernel()(xs_n, asrc_tt, adst_tt, g, esrc, edst)

        out = pl.pallas_call(
            _d_body,
            grid=(N // BLKD,),
            in_specs=[
                pl.BlockSpec((H, NREL, BLKD, C), lambda i: (0, 0, i, 0)),
                pl.BlockSpec((H, NREL, BLKD, NSUB), lambda i: (0, 0, i, 0)),
                pl.BlockSpec((H, BLKD, C), lambda i: (0, i, 0)),
                pl.BlockSpec((BLKD, H), lambda i: (i, 0)),
                pl.BlockSpec((1, H * C), lambda i: (0, 0)),
            ],
            out_specs=pl.BlockSpec((BLKD, H * C), lambda i: (i, 0)),
            out_shape=jax.ShapeDtypeStruct((N, H * C), jnp.float32),
        )(num, den[:, :, :, :N].transpose(0, 1, 3, 2), xs_n, wself[:N],
          bias.astype(jnp.float32).reshape(1, H * C))

    return out


# speedup vs baseline: 51.1258x; 1.0819x over previous
import dataclasses
import functools

import jax
import jax.numpy as jnp
from jax import lax
from jax.experimental import pallas as pl
from jax.experimental.pallas import tpu as pltpu
from jax.experimental.pallas import tpu_sc as plsc

N = 10000
D = 128
H = 2
C = 128
NREL = 5
E = 320000

NPAD = 10240
NSUB = 16
EB = 64
CHUNKS_PER_SUB = 314
E_PAD = CHUNKS_PER_SUB * NSUB * EB
N_ACC = 10240
ROWS_PER_SUB = 640
LAST_ROWS = 400
BLKA = 256
BLKD = 200


def _lrelu(v):
    return jnp.where(v > 0.0, v, v * 0.2)


def _a1_body(x_ref, ws_ref, wd_ref, atts_ref, attd_ref,
             xs_out_ref, asrc_ref, adst_ref):
    x = x_ref[...]
    xs = jnp.dot(x, ws_ref[...], preferred_element_type=jnp.float32)
    xd = jnp.dot(x, wd_ref[...], preferred_element_type=jnp.float32)
    a_s, a_d = [], []
    for h in range(H):
        xs_h = xs[:, h * C:(h + 1) * C]
        xd_h = xd[:, h * C:(h + 1) * C]
        xs_out_ref[h] = xs_h
        a_s.append(jnp.sum(xs_h * atts_ref[h][None, :], axis=-1, keepdims=True))
        a_d.append(jnp.sum(xd_h * attd_ref[h][None, :], axis=-1, keepdims=True))
    asrc_ref[...] = jnp.concatenate(a_s, axis=1)
    adst_ref[...] = jnp.concatenate(a_d, axis=1)


def _a2_body(asrc_ref, adst_ref, g_ref, wself_ref):
    a_s = asrc_ref[...]
    a_d = adst_ref[...]
    g = _lrelu(jnp.max(a_s, axis=0, keepdims=True)
               + jnp.max(a_d, axis=0, keepdims=True))
    g_ref[...] = jnp.broadcast_to(g.T, (H, 16))
    wself_ref[...] = jnp.exp(_lrelu(a_s + a_d) - g)


def _d_body(num_ref, den_ref, xs_ref, wself_ref, bias_ref, out_ref):
    for h in range(H):
        xsh = xs_ref[h]
        ws = wself_ref[:, h:h + 1]
        acc = jnp.zeros((BLKD, C), jnp.float32)
        for r in range(NREL):
            numer = num_ref[h, r] + ws * xsh
            den = (jnp.sum(den_ref[h, r], axis=-1)[:, None] + ws + 1e-16)
            acc = acc + numer / den
        out_ref[:, h * C:(h + 1) * C] = acc + bias_ref[0, h * C:(h + 1) * C][None, :]


def _bcast_lane(v, j):
    idx = jnp.full((16, 1), j, jnp.int32)
    dnums = lax.GatherDimensionNumbers(
        offset_dims=(), collapsed_slice_dims=(0,), start_index_map=(0,))
    return lax.gather(v, idx, dnums, (1,),
                      mode=lax.GatherScatterMode.PROMISE_IN_BOUNDS)


def _zero_rowbuf(rowbuf):
    @pl.loop(0, EB)
    def _zz(i):
        i = i.astype(jnp.int32)
        for cc in range(C // 16):
            rowbuf.at[i, pl.ds(cc * 16, 16)][...] = jnp.zeros((16,), jnp.float32)


def _proc_chunk(srcbuf, dstbuf, rowbuf, asrc_tab, adst_tab, den_tab, gv, acc):
    @pl.loop(0, EB, step=16)
    def _grp(i):
        i = i.astype(jnp.int32)
        idx_s = srcbuf[pl.ds(i, 16)]
        idx_d = dstbuf[pl.ds(i, 16)]
        a_s = plsc.load_gather(asrc_tab, [idx_s])
        a_d = plsc.load_gather(adst_tab, [idx_d])
        w = jnp.exp(_lrelu(a_s + a_d) - gv)
        for j in range(16):
            wj = _bcast_lane(w, j)
            for cc in range(C // 16):
                sl = pl.ds(cc * 16, 16)
                v = rowbuf[i + j, sl]
                rowbuf.at[i + j, sl][...] = v * wj
            dj = _bcast_lane(idx_d, j)
            dv = plsc.load_gather(den_tab, [dj])
            plsc.store_scatter(den_tab, [dj], dv + wj)

    pltpu.sync_copy(rowbuf, acc.at[dstbuf], add=True)


def _sc_body(xs_hbm, asrc_hbm, adst_hbm, g_hbm, esrc_hbm, edst_hbm,
             out_hbm, den_hbm,
             asrc_tab, adst_tab, g_tab, den_tab,
             srcbuf0, dstbuf0, rowbuf0, srcbuf1, dstbuf1, rowbuf1,
             acc, sem0, sem1):
    h = lax.axis_index("c").astype(jnp.int32)
    sid = lax.axis_index("s").astype(jnp.int32)
    rows_per_sub = jnp.int32(ROWS_PER_SUB)
    chunks_per_sub = jnp.int32(CHUNKS_PER_SUB)

    pltpu.sync_copy(asrc_hbm.at[h], asrc_tab)
    pltpu.sync_copy(adst_hbm.at[h], adst_tab)
    pltpu.sync_copy(g_hbm.at[h], g_tab)

    xs_h = xs_hbm.at[h]

    for r in range(NREL):
        _zero_rowbuf(rowbuf0)

        @pl.loop(0, N_ACC, step=16)
        def _zd(i):
            i = i.astype(jnp.int32)
            den_tab.at[pl.ds(i, 16)][...] = jnp.zeros((16,), jnp.float32)

        @pl.loop(0, ROWS_PER_SUB, step=EB)
        def _z(i):
            i = i.astype(jnp.int32)
            pltpu.sync_copy(rowbuf0, acc.at[pl.ds(sid * rows_per_sub + i, EB)])

        plsc.subcore_barrier()

        gv = g_tab[pl.ds(0, 16)]
        base = sid * chunks_per_sub

        pltpu.sync_copy(esrc_hbm.at[r, base], srcbuf0)
        pltpu.sync_copy(edst_hbm.at[r, base], dstbuf0)
        cp0 = pltpu.make_async_copy(xs_h.at[srcbuf0], rowbuf0, sem0)
        cp0.start()

        @pl.loop(0, CHUNKS_PER_SUB, step=2)
        def _chunk(t):
            t = t.astype(jnp.int32)
            row = base + t

            pltpu.sync_copy(esrc_hbm.at[r, row + 1], srcbuf1)
            pltpu.sync_copy(edst_hbm.at[r, row + 1], dstbuf1)
            pltpu.make_async_copy(xs_h.at[srcbuf1], rowbuf1, sem1).start()

            pltpu.make_async_copy(xs_h.at[srcbuf0], rowbuf0, sem0).wait()
            _proc_chunk(srcbuf0, dstbuf0, rowbuf0, asrc_tab, adst_tab, den_tab, gv, acc)

            @pl.when(t + 2 < chunks_per_sub)
            def _pf():
                pltpu.sync_copy(esrc_hbm.at[r, row + 2], srcbuf0)
                pltpu.sync_copy(edst_hbm.at[r, row + 2], dstbuf0)
                pltpu.make_async_copy(xs_h.at[srcbuf0], rowbuf0, sem0).start()

            pltpu.make_async_copy(xs_h.at[srcbuf1], rowbuf1, sem1).wait()
            _proc_chunk(srcbuf1, dstbuf1, rowbuf1, asrc_tab, adst_tab, den_tab, gv, acc)

        plsc.subcore_barrier()

        @pl.when(sid < NSUB - 1)
        def _d1():
            pltpu.sync_copy(acc.at[pl.ds(sid * rows_per_sub, ROWS_PER_SUB)],
                            out_hbm.at[h, r, pl.ds(sid * rows_per_sub, ROWS_PER_SUB)])

        @pl.when(sid == NSUB - 1)
        def _d2():
            pltpu.sync_copy(acc.at[pl.ds((NSUB - 1) * ROWS_PER_SUB, LAST_ROWS)],
                            out_hbm.at[h, r, pl.ds((NSUB - 1) * ROWS_PER_SUB, LAST_ROWS)])

        pltpu.sync_copy(den_tab, den_hbm.at[h, r, sid])

        plsc.subcore_barrier()


def _make_sc_kernel():
    mesh = plsc.VectorSubcoreMesh(core_axis_name="c", subcore_axis_name="s")
    cp = pltpu.CompilerParams()
    if "needs_layout_passes" in pltpu.CompilerParams.__dataclass_fields__:
        cp = dataclasses.replace(cp, needs_layout_passes=False)
    return functools.partial(
        pl.kernel,
        mesh=mesh,
        compiler_params=cp,
        out_type=[
            jax.ShapeDtypeStruct((H, NREL, N, C), jnp.float32),
            jax.ShapeDtypeStruct((H, NREL, NSUB, N_ACC), jnp.float32),
        ],
        scratch_types=[
            pltpu.VMEM((N + 16,), jnp.float32),
            pltpu.VMEM((N + 16,), jnp.float32),
            pltpu.VMEM((16,), jnp.float32),
            pltpu.VMEM((N_ACC,), jnp.float32),
            pltpu.VMEM((EB,), jnp.int32),
            pltpu.VMEM((EB,), jnp.int32),
            pltpu.VMEM((EB, C), jnp.float32),
            pltpu.VMEM((EB,), jnp.int32),
            pltpu.VMEM((EB,), jnp.int32),
            pltpu.VMEM((EB, C), jnp.float32),
            pltpu.VMEM_SHARED((N_ACC, C), jnp.float32),
            pltpu.SemaphoreType.DMA,
            pltpu.SemaphoreType.DMA,
        ],
    )(_sc_body)


def kernel(x_capec, edge_index_parentof, edge_index_childof, edge_index_canprecede,
           edge_index_canfollow, edge_index_peerof, W_src, W_dst, att_src, att_dst, bias):
    import jax._src.config as _jcfg
    with _jcfg.enable_x64(False):
        x = x_capec.astype(jnp.float32)
        xpad = jnp.pad(x, ((0, NPAD - N), (0, 0)))

        a1 = pl.pallas_call(
            _a1_body,
            grid=(NPAD // BLKA,),
            in_specs=[
                pl.BlockSpec((BLKA, D), lambda i: (i, 0)),
                pl.BlockSpec((D, H * C), lambda i: (0, 0)),
                pl.BlockSpec((D, H * C), lambda i: (0, 0)),
                pl.BlockSpec((H, C), lambda i: (0, 0)),
                pl.BlockSpec((H, C), lambda i: (0, 0)),
            ],
            out_specs=[
                pl.BlockSpec((H, BLKA, C), lambda i: (0, i, 0)),
                pl.BlockSpec((BLKA, H), lambda i: (i, 0)),
                pl.BlockSpec((BLKA, H), lambda i: (i, 0)),
            ],
            out_shape=[
                jax.ShapeDtypeStruct((H, NPAD, C), jnp.float32),
                jax.ShapeDtypeStruct((NPAD, H), jnp.float32),
                jax.ShapeDtypeStruct((NPAD, H), jnp.float32),
            ],
        )
        xs_t, a_src, a_dst = a1(xpad, W_src.astype(jnp.float32), W_dst.astype(jnp.float32),
                                att_src.astype(jnp.float32), att_dst.astype(jnp.float32))

        a2 = pl.pallas_call(
            _a2_body,
            out_shape=[
                jax.ShapeDtypeStruct((H, 16), jnp.float32),
                jax.ShapeDtypeStruct((NPAD, H), jnp.float32),
            ],
        )
        g, wself = a2(a_src, a_dst)

        xs_n = xs_t[:, :N, :]
        pad_tab = ((0, 0), (0, 16))
        asrc_tt = jnp.pad(a_src[:N].T, pad_tab)
        adst_tt = jnp.pad(a_dst[:N].T, pad_tab)

        edges = [edge_index_parentof, edge_index_childof, edge_index_canprecede,
                 edge_index_canfollow, edge_index_peerof]
        esrc = jnp.stack([e[0] for e in edges]).astype(jnp.int32)
        edst = jnp.stack([e[1] for e in edges]).astype(jnp.int32)
        esrc = jnp.pad(esrc, ((0, 0), (0, E_PAD - E)), constant_values=0)
        edst = jnp.pad(edst, ((0, 0), (0, E_PAD - E)), constant_values=N)
        esrc = esrc.reshape(NREL, E_PAD // EB, EB)
        edst = edst.reshape(NREL, E_PAD // EB, EB)

        num, den = _make_sc_kernel()(xs_n, asrc_tt, adst_tt, g, esrc, edst)

        out = pl.pallas_call(
            _d_body,
            grid=(N // BLKD,),
            in_specs=[
                pl.BlockSpec((H, NREL, BLKD, C), lambda i: (0, 0, i, 0)),
                pl.BlockSpec((H, NREL, BLKD, NSUB), lambda i: (0, 0, i, 0)),
                pl.BlockSpec((H, BLKD, C), lambda i: (0, i, 0)),
                pl.BlockSpec((BLKD, H), lambda i: (i, 0)),
                pl.BlockSpec((1, H * C), lambda i: (0, 0)),
            ],
            out_specs=pl.BlockSpec((BLKD, H * C), lambda i: (i, 0)),
            out_shape=jax.ShapeDtypeStruct((N, H * C), jnp.float32),
        )(num, den[:, :, :, :N].transpose(0, 1, 3, 2), xs_n, wself[:N],
          bias.astype(jnp.float32).reshape(1, H * C))

    return out
